# Initial kernel scaffold; baseline (speedup 1.0000x reference)
import jax
import jax.numpy as jnp
from jax.experimental import pallas as pl


def kernel(S, pe_weight):
    raise NotImplementedError("write your pallas kernel here")


# speedup vs baseline: 2.9215x; 2.9215x over previous
import jax
import jax.numpy as jnp
from jax.experimental import pallas as pl

_BT = 512


def _add_body(s_ref, pe_ref, o_ref):
    o_ref[...] = s_ref[...] + pe_ref[...][None, :, :]


def kernel(S, pe_weight):
    B, T, D = S.shape
    grid = (T // _BT, B)
    return pl.pallas_call(
        _add_body,
        grid=grid,
        in_specs=[
            pl.BlockSpec((1, _BT, D), lambda t, b: (b, t, 0)),
            pl.BlockSpec((_BT, D), lambda t, b: (t, 0)),
        ],
        out_specs=pl.BlockSpec((1, _BT, D), lambda t, b: (b, t, 0)),
        out_shape=jax.ShapeDtypeStruct((B, T, D), S.dtype),
    )(S, pe_weight)
